# baseline (speedup 1.0000x reference)
import jax
import jax.numpy as jnp
from jax import lax
from jax.experimental import pallas as pl
from jax.experimental.pallas import tpu as pltpu
from jax.experimental.pallas import tpu_sc as plsc

NUM_NODES = 100000
D = 128
B = 16384
S = 51
P = S - 1
PW = 64
NC, NS = 2, 16
NW = NC * NS
RW = B // NW
G = 8
NCHUNK = RW // G
SPLITS = ((0, 128), (128, 128), (256, 128), (384, 24))
EPS = 1e-10
NLANES = 16
NCHK = D // NLANES


def _sc_body(idx_hbm, table_hbm, gam_hbm,
             idx_v0, idx_v1, rows_v0, rows_v1,
             outg_v0, outg_v1,
             sem_i0, sem_i1, sem_g0, sem_g1, sem_o0, sem_o1):
    idx_v = (idx_v0, idx_v1)
    rows_v = (rows_v0, rows_v1)
    outg_v = (outg_v0, outg_v1)
    sem_i = (sem_i0, sem_i1)
    sem_g = (sem_g0, sem_g1)
    sem_o = (sem_o0, sem_o1)

    wid = lax.axis_index("s") * NC + lax.axis_index("c")
    row0 = wid * RW

    lane = lax.iota(jnp.int32, NLANES)
    zero16 = jnp.zeros((NLANES,), jnp.float32)
    perms = [lane ^ sh for sh in (1, 2, 4, 8)]

    def hsum(v):
        for p in perms:
            v = v + jnp.take(v, p)
        return v

    bitmasks = [(lane >> k) % 2 == 1 for k in range(4)]

    def tree16(A):
        V = list(A)
        for k in range(4):
            m = bitmasks[k]
            p = perms[k]
            V = [jnp.where(m, V[2 * j + 1], V[2 * j])
                 + jnp.take(jnp.where(m, V[2 * j], V[2 * j + 1]), p)
                 for j in range(len(V) // 2)]
        return V[0]

    def idx_src(ci):
        return idx_hbm.at[pl.ds((row0 + ci * G) * S, G * S)]

    def fire_idx(ci, b):
        pltpu.make_async_copy(idx_src(ci), idx_v[b], sem_i[b]).start()

    def wait_idx(b):
        pltpu.make_async_copy(idx_src(0), idx_v[b], sem_i[b]).wait()

    def fire_gather(b):
        for off, n in SPLITS:
            pltpu.make_async_copy(
                table_hbm.at[idx_v[b].at[pl.ds(off, n)]],
                rows_v[b].at[pl.ds(off, n)], sem_g[b]).start()

    def wait_gather(b):
        for off, n in SPLITS:
            pltpu.make_async_copy(
                table_hbm.at[idx_v[b].at[pl.ds(off, n)]],
                rows_v[b].at[pl.ds(off, n)], sem_g[b]).wait()

    def out_slice(ci):
        o = (row0 + ci * G) * PW
        return gam_hbm.at[pl.ds(o, G * PW)]

    def fire_out(ci, b):
        pltpu.make_async_copy(outg_v[b], out_slice(ci), sem_o[b]).start()

    def wait_out(b):
        pltpu.make_async_copy(outg_v[b], out_slice(0), sem_o[b]).wait()

    def compute(b):
        rows = rows_v[b]
        outg = outg_v[b]

        def pair_accs(r, s):
            v = [rows[r * S + s, pl.ds(c * NLANES, NLANES)]
                 for c in range(NCHK)]
            d0 = u_regs[0] * v[0]
            d1 = u_regs[4] * v[4]
            n0 = v[0] * v[0]
            n1 = v[4] * v[4]
            for c in range(1, 4):
                d0 = d0 + u_regs[c] * v[c]
                d1 = d1 + u_regs[c + 4] * v[c + 4]
                n0 = n0 + v[c] * v[c]
                n1 = n1 + v[c + 4] * v[c + 4]
            return d0 + d1, n0 + n1

        def row_body(r, carry):
            nonlocal u_regs, gamma_vec
            u_regs = [rows[r * S, pl.ds(c * NLANES, NLANES)]
                      for c in range(NCHK)]
            acc0 = u_regs[0] * u_regs[0]
            acc1 = u_regs[4] * u_regs[4]
            for c in range(1, 4):
                acc0 = acc0 + u_regs[c] * u_regs[c]
                acc1 = acc1 + u_regs[c + 4] * u_regs[c + 4]
            nu_splat = hsum(acc0 + acc1)
            alpha = jnp.maximum(1.0 - nu_splat, EPS)

            def gamma_vec(dotv, nvv):
                sqd = nu_splat + nvv - 2.0 * dotv
                beta = jnp.maximum(1.0 - nvv, EPS)
                gam = 1.0 + 2.0 * sqd / (alpha * beta)
                return jnp.maximum(gam, 1.0 + 1e-7)

            for g in range(3):
                s0 = NLANES * g + 1
                Ad, An = [], []
                for k in range(NLANES):
                    d, n = pair_accs(r, s0 + k)
                    Ad.append(d)
                    An.append(n)
                o = r * PW + NLANES * g
                outg[pl.ds(o, NLANES)] = gamma_vec(tree16(Ad), tree16(An))

            d49, n49 = pair_accs(r, 49)
            d50, n50 = pair_accs(r, 50)
            Ad = [d49, d50] + [zero16] * 14
            An = [n49, n50] + [zero16] * 14
            o = r * PW + NLANES * 3
            outg[pl.ds(o, NLANES)] = gamma_vec(tree16(Ad), tree16(An))
            return carry

        u_regs = None
        gamma_vec = None
        lax.fori_loop(0, G, row_body, 0)

    fire_idx(0, 0)
    fire_idx(1, 1)
    wait_idx(0)
    fire_gather(0)

    def step(gi, carry):
        for b in range(2):
            i = 2 * gi + b
            wait_gather(b)
            fire_idx(jnp.minimum(i + 2, NCHUNK - 1), b)
            wait_idx(b ^ 1)
            fire_gather(b ^ 1)

            @pl.when(i >= 2)
            def _():
                wait_out(b)

            compute(b)
            fire_out(i, b)
        return carry

    lax.fori_loop(0, NCHUNK // 2, step, 0)
    wait_gather(NCHUNK % 2)
    wait_idx((NCHUNK + 1) % 2)
    wait_out(0)
    wait_out(1)


_SC_CALL_CACHE = {}


def _sc_call(idx, table):
    if "call" not in _SC_CALL_CACHE:
        _SC_CALL_CACHE["call"] = pl.kernel(
            _sc_body,
            out_type=jax.ShapeDtypeStruct((B * PW,), jnp.float32),
            mesh=plsc.VectorSubcoreMesh(
                core_axis_name="c", subcore_axis_name="s",
                num_cores=NC, num_subcores=NS),
            scratch_types=[
                pltpu.VMEM((G * S,), jnp.int32),
                pltpu.VMEM((G * S,), jnp.int32),
                pltpu.VMEM((G * S, D), jnp.float32),
                pltpu.VMEM((G * S, D), jnp.float32),
                pltpu.VMEM((G * PW,), jnp.float32),
                pltpu.VMEM((G * PW,), jnp.float32),
                pltpu.SemaphoreType.DMA,
                pltpu.SemaphoreType.DMA,
                pltpu.SemaphoreType.DMA,
                pltpu.SemaphoreType.DMA,
                pltpu.SemaphoreType.DMA,
                pltpu.SemaphoreType.DMA,
            ],
        )
    return _SC_CALL_CACHE["call"](idx, table)

RB = 1024


def _tc_body(gam_ref, out_ref):
    pid = pl.program_id(0)
    gamma = gam_ref[...]
    lane = lax.broadcasted_iota(jnp.int32, gamma.shape, 1)
    mask = lane < P
    dist = -jnp.log(gamma + jnp.sqrt((gamma - 1.0) * (gamma + 1.0)))
    dist = jnp.where(mask, dist, -jnp.inf)
    m = jnp.max(dist, axis=1, keepdims=True)
    e = jnp.where(mask, jnp.exp(dist - m), 0.0)
    lse = jnp.log(jnp.sum(e, axis=1, keepdims=True)) + m
    partial = jnp.sum(lse - dist[:, 0:1], keepdims=True)

    @pl.when(pid == 0)
    def _():
        out_ref[...] = jnp.zeros((1, 1), jnp.float32)

    out_ref[...] += partial * (1.0 / B)


def kernel(data, table):
    idx = data.reshape(-1).astype(jnp.int32)
    gam = _sc_call(idx, table).reshape(B, PW)
    loss = pl.pallas_call(
        _tc_body,
        grid=(B // RB,),
        in_specs=[
            pl.BlockSpec((RB, PW), lambda i: (i, 0)),
        ],
        out_specs=pl.BlockSpec((1, 1), lambda i: (0, 0)),
        out_shape=jax.ShapeDtypeStruct((1, 1), jnp.float32),
    )(gam)
    return loss[0, 0]


# speedup vs baseline: 1.0659x; 1.0659x over previous
import jax
import jax.numpy as jnp
from jax import lax
from jax.experimental import pallas as pl
from jax.experimental.pallas import tpu as pltpu
from jax.experimental.pallas import tpu_sc as plsc

NUM_NODES = 100000
D = 128
B = 16384
S = 51
P = S - 1
PW = 64
NC, NS = 2, 16
NW = NC * NS
RW = B // NW
G = 8
NCHUNK = RW // G
SPLITS = ((0, 128), (128, 128), (256, 128), (384, 24))
EPS = 1e-10
NLANES = 16
NCHK = D // NLANES


def _sc_body(idx_hbm, table_hbm, dot_hbm, nv_hbm,
             idx_v0, idx_v1, rows_v0, rows_v1,
             outd_v0, outd_v1, outn_v0, outn_v1,
             sem_i0, sem_i1, sem_g0, sem_g1, sem_o0, sem_o1):
    idx_v = (idx_v0, idx_v1)
    rows_v = (rows_v0, rows_v1)
    outd_v = (outd_v0, outd_v1)
    outn_v = (outn_v0, outn_v1)
    sem_i = (sem_i0, sem_i1)
    sem_g = (sem_g0, sem_g1)
    sem_o = (sem_o0, sem_o1)

    wid = lax.axis_index("s") * NC + lax.axis_index("c")
    row0 = wid * RW

    lane = lax.iota(jnp.int32, NLANES)
    zero16 = jnp.zeros((NLANES,), jnp.float32)
    perms = [lane ^ sh for sh in (1, 2, 4, 8)]

    def hsum(v):
        for p in perms:
            v = v + jnp.take(v, p)
        return v

    bitmasks = [(lane >> k) % 2 == 1 for k in range(4)]

    def tree16(A):
        V = list(A)
        for k in range(4):
            m = bitmasks[k]
            p = perms[k]
            V = [jnp.where(m, V[2 * j + 1], V[2 * j])
                 + jnp.take(jnp.where(m, V[2 * j], V[2 * j + 1]), p)
                 for j in range(len(V) // 2)]
        return V[0]

    def idx_src(ci):
        return idx_hbm.at[pl.ds((row0 + ci * G) * S, G * S)]

    def fire_idx(ci, b):
        pltpu.make_async_copy(idx_src(ci), idx_v[b], sem_i[b]).start()

    def wait_idx(b):
        pltpu.make_async_copy(idx_src(0), idx_v[b], sem_i[b]).wait()

    def fire_gather(b):
        for off, n in SPLITS:
            pltpu.make_async_copy(
                table_hbm.at[idx_v[b].at[pl.ds(off, n)]],
                rows_v[b].at[pl.ds(off, n)], sem_g[b]).start()

    def wait_gather(b):
        for off, n in SPLITS:
            pltpu.make_async_copy(
                table_hbm.at[idx_v[b].at[pl.ds(off, n)]],
                rows_v[b].at[pl.ds(off, n)], sem_g[b]).wait()

    def out_slices(ci):
        o = (row0 + ci * G) * PW
        return (dot_hbm.at[pl.ds(o, G * PW)], nv_hbm.at[pl.ds(o, G * PW)])

    def fire_out(ci, b):
        dslc, nslc = out_slices(ci)
        pltpu.make_async_copy(outd_v[b], dslc, sem_o[b]).start()
        pltpu.make_async_copy(outn_v[b], nslc, sem_o[b]).start()

    def wait_out(b):
        dslc, nslc = out_slices(0)
        pltpu.make_async_copy(outd_v[b], dslc, sem_o[b]).wait()
        pltpu.make_async_copy(outn_v[b], nslc, sem_o[b]).wait()

    def compute(b):
        rows = rows_v[b]
        outd = outd_v[b]
        outn = outn_v[b]

        def pair_accs(r, s):
            v = [rows[r * S + s, pl.ds(c * NLANES, NLANES)]
                 for c in range(NCHK)]
            d0 = u_regs[0] * v[0]
            d1 = u_regs[4] * v[4]
            n0 = v[0] * v[0]
            n1 = v[4] * v[4]
            for c in range(1, 4):
                d0 = d0 + u_regs[c] * v[c]
                d1 = d1 + u_regs[c + 4] * v[c + 4]
                n0 = n0 + v[c] * v[c]
                n1 = n1 + v[c + 4] * v[c + 4]
            return d0 + d1, n0 + n1

        def row_body(r, carry):
            nonlocal u_regs
            u_regs = [rows[r * S, pl.ds(c * NLANES, NLANES)]
                      for c in range(NCHK)]
            acc0 = u_regs[0] * u_regs[0]
            acc1 = u_regs[4] * u_regs[4]
            for c in range(1, 4):
                acc0 = acc0 + u_regs[c] * u_regs[c]
                acc1 = acc1 + u_regs[c + 4] * u_regs[c + 4]
            accu = acc0 + acc1

            def grp_body(g, carry2):
                s0 = NLANES * g + 1
                Ad, An = [], []
                for k in range(NLANES):
                    d, n = pair_accs(r, s0 + k)
                    Ad.append(d)
                    An.append(n)
                o = r * PW + NLANES * g
                outd[pl.ds(o, NLANES)] = tree16(Ad)
                outn[pl.ds(o, NLANES)] = tree16(An)
                return carry2

            lax.fori_loop(0, 3, grp_body, 0)

            d49, n49 = pair_accs(r, 49)
            d50, n50 = pair_accs(r, 50)
            Ad = [d49, d50, accu] + [zero16] * 13
            An = [n49, n50] + [zero16] * 14
            o = r * PW + NLANES * 3
            outd[pl.ds(o, NLANES)] = tree16(Ad)
            outn[pl.ds(o, NLANES)] = tree16(An)
            return carry

        u_regs = None
        lax.fori_loop(0, G, row_body, 0)

    fire_idx(0, 0)
    fire_idx(1, 1)
    wait_idx(0)
    fire_gather(0)

    def step(gi, carry):
        for b in range(2):
            i = 2 * gi + b
            wait_gather(b)
            fire_idx(jnp.minimum(i + 2, NCHUNK - 1), b)
            wait_idx(b ^ 1)
            fire_gather(b ^ 1)

            @pl.when(i >= 2)
            def _():
                wait_out(b)

            compute(b)
            fire_out(i, b)
        return carry

    lax.fori_loop(0, NCHUNK // 2, step, 0)
    wait_gather(NCHUNK % 2)
    wait_idx((NCHUNK + 1) % 2)
    wait_out(0)
    wait_out(1)


_SC_CALL_CACHE = {}


def _sc_call(idx, table):
    if "call" not in _SC_CALL_CACHE:
        _SC_CALL_CACHE["call"] = pl.kernel(
            _sc_body,
            out_type=[
                jax.ShapeDtypeStruct((B * PW,), jnp.float32),
                jax.ShapeDtypeStruct((B * PW,), jnp.float32),
            ],
            mesh=plsc.VectorSubcoreMesh(
                core_axis_name="c", subcore_axis_name="s",
                num_cores=NC, num_subcores=NS),
            scratch_types=[
                pltpu.VMEM((G * S,), jnp.int32),
                pltpu.VMEM((G * S,), jnp.int32),
                pltpu.VMEM((G * S, D), jnp.float32),
                pltpu.VMEM((G * S, D), jnp.float32),
                pltpu.VMEM((G * PW,), jnp.float32),
                pltpu.VMEM((G * PW,), jnp.float32),
                pltpu.VMEM((G * PW,), jnp.float32),
                pltpu.VMEM((G * PW,), jnp.float32),
                pltpu.SemaphoreType.DMA,
                pltpu.SemaphoreType.DMA,
                pltpu.SemaphoreType.DMA,
                pltpu.SemaphoreType.DMA,
                pltpu.SemaphoreType.DMA,
                pltpu.SemaphoreType.DMA,
            ],
        )
    return _SC_CALL_CACHE["call"](idx, table)

RB = 1024


def _tc_body(dot_ref, nv_ref, out_ref):
    pid = pl.program_id(0)
    dotb = dot_ref[...]
    nvb = nv_ref[...]
    nu = dotb[:, P:P + 1]
    lane = lax.broadcasted_iota(jnp.int32, dotb.shape, 1)
    mask = lane < P
    sq_dist = nu + nvb - 2.0 * dotb
    alpha = jnp.maximum(1.0 - nu, EPS)
    beta = jnp.maximum(1.0 - nvb, EPS)
    gamma = 1.0 + 2.0 * sq_dist / (alpha * beta)
    gamma = jnp.maximum(gamma, 1.0 + 1e-7)
    dist = -jnp.log(gamma + jnp.sqrt((gamma - 1.0) * (gamma + 1.0)))
    dist = jnp.where(mask, dist, -jnp.inf)
    m = jnp.max(dist, axis=1, keepdims=True)
    e = jnp.where(mask, jnp.exp(dist - m), 0.0)
    lse = jnp.log(jnp.sum(e, axis=1, keepdims=True)) + m
    partial = jnp.sum(lse - dist[:, 0:1], keepdims=True)

    @pl.when(pid == 0)
    def _():
        out_ref[...] = jnp.zeros((1, 1), jnp.float32)

    out_ref[...] += partial * (1.0 / B)


def kernel(data, table):
    idx = data.reshape(-1).astype(jnp.int32)
    dot, nv = _sc_call(idx, table)
    dot = dot.reshape(B, PW)
    nv = nv.reshape(B, PW)
    loss = pl.pallas_call(
        _tc_body,
        grid=(B // RB,),
        in_specs=[
            pl.BlockSpec((RB, PW), lambda i: (i, 0)),
            pl.BlockSpec((RB, PW), lambda i: (i, 0)),
        ],
        out_specs=pl.BlockSpec((1, 1), lambda i: (0, 0)),
        out_shape=jax.ShapeDtypeStruct((1, 1), jnp.float32),
    )(dot, nv)
    return loss[0, 0]


# speedup vs baseline: 1.0787x; 1.0120x over previous
import jax
import jax.numpy as jnp
from jax import lax
from jax.experimental import pallas as pl
from jax.experimental.pallas import tpu as pltpu
from jax.experimental.pallas import tpu_sc as plsc

NUM_NODES = 100000
D = 128
B = 16384
S = 51
P = S - 1
PW = 64
NC, NS = 2, 16
NW = NC * NS
RW = B // NW
G = 8
NCHUNK = RW // G
SPLITS = ((0, 128), (128, 128), (256, 128), (384, 24))
EPS = 1e-10
NLANES = 16
NCHK = D // NLANES


def _sc_body(idx_hbm, table_hbm, dot_hbm, nv_hbm,
             idx_v0, idx_v1, rows_v0, rows_v1,
             outd_v0, outd_v1, outn_v0, outn_v1,
             sem_i0, sem_i1, sem_g0, sem_g1, sem_o0, sem_o1):
    idx_v = (idx_v0, idx_v1)
    rows_v = (rows_v0, rows_v1)
    outd_v = (outd_v0, outd_v1)
    outn_v = (outn_v0, outn_v1)
    sem_i = (sem_i0, sem_i1)
    sem_g = (sem_g0, sem_g1)
    sem_o = (sem_o0, sem_o1)

    wid = lax.axis_index("s") * NC + lax.axis_index("c")
    row0 = wid * RW

    lane = lax.iota(jnp.int32, NLANES)
    zero16 = jnp.zeros((NLANES,), jnp.float32)
    perms = [lane ^ sh for sh in (1, 2, 4, 8)]

    def hsum(v):
        for p in perms:
            v = v + jnp.take(v, p)
        return v

    bitmasks = [(lane >> k) % 2 == 1 for k in range(4)]

    def tree16(A):
        V = list(A)
        for k in range(4):
            m = bitmasks[k]
            p = perms[k]
            V = [jnp.where(m, V[2 * j + 1], V[2 * j])
                 + jnp.take(jnp.where(m, V[2 * j], V[2 * j + 1]), p)
                 for j in range(len(V) // 2)]
        return V[0]

    def idx_src(ci):
        return idx_hbm.at[pl.ds((row0 + ci * G) * S, G * S)]

    def fire_idx(ci, b):
        pltpu.make_async_copy(idx_src(ci), idx_v[b], sem_i[b]).start()

    def wait_idx(b):
        pltpu.make_async_copy(idx_src(0), idx_v[b], sem_i[b]).wait()

    def fire_gather(b):
        for off, n in SPLITS:
            pltpu.make_async_copy(
                table_hbm.at[idx_v[b].at[pl.ds(off, n)]],
                rows_v[b].at[pl.ds(off, n)], sem_g[b]).start()

    def wait_gather(b):
        for off, n in SPLITS:
            pltpu.make_async_copy(
                table_hbm.at[idx_v[b].at[pl.ds(off, n)]],
                rows_v[b].at[pl.ds(off, n)], sem_g[b]).wait()

    def out_slices(ci):
        o = (row0 + ci * G) * PW
        return (dot_hbm.at[pl.ds(o, G * PW)], nv_hbm.at[pl.ds(o, G * PW)])

    def fire_out(ci, b):
        dslc, nslc = out_slices(ci)
        pltpu.make_async_copy(outd_v[b], dslc, sem_o[b]).start()
        pltpu.make_async_copy(outn_v[b], nslc, sem_o[b]).start()

    def wait_out(b):
        dslc, nslc = out_slices(0)
        pltpu.make_async_copy(outd_v[b], dslc, sem_o[b]).wait()
        pltpu.make_async_copy(outn_v[b], nslc, sem_o[b]).wait()

    def compute(b):
        rows = rows_v[b]
        outd = outd_v[b]
        outn = outn_v[b]

        def pair_accs(r, s):
            v = [rows[r * S + s, pl.ds(c * NLANES, NLANES)]
                 for c in range(NCHK)]
            d0 = u_regs[0] * v[0]
            d1 = u_regs[4] * v[4]
            n0 = v[0] * v[0]
            n1 = v[4] * v[4]
            for c in range(1, 4):
                d0 = d0 + u_regs[c] * v[c]
                d1 = d1 + u_regs[c + 4] * v[c + 4]
                n0 = n0 + v[c] * v[c]
                n1 = n1 + v[c + 4] * v[c + 4]
            return d0 + d1, n0 + n1

        def row_body(r, carry):
            nonlocal u_regs
            u_regs = [rows[r * S, pl.ds(c * NLANES, NLANES)]
                      for c in range(NCHK)]
            acc0 = u_regs[0] * u_regs[0]
            acc1 = u_regs[4] * u_regs[4]
            for c in range(1, 4):
                acc0 = acc0 + u_regs[c] * u_regs[c]
                acc1 = acc1 + u_regs[c + 4] * u_regs[c + 4]
            accu = acc0 + acc1

            def grp_body(g, carry2):
                s0 = NLANES * g + 1
                Ad, An = [], []
                for k in range(NLANES):
                    d, n = pair_accs(r, s0 + k)
                    Ad.append(d)
                    An.append(n)
                o = r * PW + NLANES * g
                outd[pl.ds(o, NLANES)] = tree16(Ad)
                outn[pl.ds(o, NLANES)] = tree16(An)
                return carry2

            lax.fori_loop(0, 3, grp_body, 0)

            d49, n49 = pair_accs(r, 49)
            d50, n50 = pair_accs(r, 50)
            Ad = [d49, d50, accu] + [zero16] * 13
            An = [n49, n50] + [zero16] * 14
            o = r * PW + NLANES * 3
            outd[pl.ds(o, NLANES)] = tree16(Ad)
            outn[pl.ds(o, NLANES)] = tree16(An)
            return carry

        u_regs = None
        lax.fori_loop(0, G, row_body, 0)

    fire_idx(0, 0)
    fire_idx(1, 1)
    wait_idx(0)
    fire_gather(0)

    def step(gi, carry):
        for b in range(2):
            i = 2 * gi + b
            wait_gather(b)
            fire_idx(jnp.minimum(i + 2, NCHUNK - 1), b)
            wait_idx(b ^ 1)
            fire_gather(b ^ 1)

            @pl.when(i >= 2)
            def _():
                wait_out(b)

            compute(b)
            fire_out(i, b)
        return carry

    lax.fori_loop(0, NCHUNK // 2, step, 0)
    wait_gather(NCHUNK % 2)
    wait_idx((NCHUNK + 1) % 2)
    wait_out(0)
    wait_out(1)


_SC_CALL_CACHE = {}


def _sc_call(idx, table):
    if "call" not in _SC_CALL_CACHE:
        _SC_CALL_CACHE["call"] = pl.kernel(
            _sc_body,
            out_type=[
                jax.ShapeDtypeStruct((B * PW,), jnp.float32),
                jax.ShapeDtypeStruct((B * PW,), jnp.float32),
            ],
            mesh=plsc.VectorSubcoreMesh(
                core_axis_name="c", subcore_axis_name="s",
                num_cores=NC, num_subcores=NS),
            scratch_types=[
                pltpu.VMEM((G * S,), jnp.int32),
                pltpu.VMEM((G * S,), jnp.int32),
                pltpu.VMEM((G * S, D), jnp.float32),
                pltpu.VMEM((G * S, D), jnp.float32),
                pltpu.VMEM((G * PW,), jnp.float32),
                pltpu.VMEM((G * PW,), jnp.float32),
                pltpu.VMEM((G * PW,), jnp.float32),
                pltpu.VMEM((G * PW,), jnp.float32),
                pltpu.SemaphoreType.DMA,
                pltpu.SemaphoreType.DMA,
                pltpu.SemaphoreType.DMA,
                pltpu.SemaphoreType.DMA,
                pltpu.SemaphoreType.DMA,
                pltpu.SemaphoreType.DMA,
            ],
        )
    return _SC_CALL_CACHE["call"](idx, table)

RB = 4096


def _tc_body(dot_ref, nv_ref, out_ref):
    pid = pl.program_id(0)
    dotb = dot_ref[...]
    nvb = nv_ref[...]
    nu = dotb[:, P:P + 1]
    lane = lax.broadcasted_iota(jnp.int32, dotb.shape, 1)
    mask = lane < P
    sq_dist = nu + nvb - 2.0 * dotb
    alpha = jnp.maximum(1.0 - nu, EPS)
    beta = jnp.maximum(1.0 - nvb, EPS)
    gamma = 1.0 + 2.0 * sq_dist / (alpha * beta)
    gamma = jnp.maximum(gamma, 1.0 + 1e-7)
    dist = -jnp.log(gamma + jnp.sqrt((gamma - 1.0) * (gamma + 1.0)))
    dist = jnp.where(mask, dist, -jnp.inf)
    m = jnp.max(dist, axis=1, keepdims=True)
    e = jnp.where(mask, jnp.exp(dist - m), 0.0)
    lse = jnp.log(jnp.sum(e, axis=1, keepdims=True)) + m
    partial = jnp.sum(lse - dist[:, 0:1], keepdims=True)

    @pl.when(pid == 0)
    def _():
        out_ref[...] = jnp.zeros((1, 1), jnp.float32)

    out_ref[...] += partial * (1.0 / B)


def kernel(data, table):
    idx = data.reshape(-1).astype(jnp.int32)
    dot, nv = _sc_call(idx, table)
    dot = dot.reshape(B, PW)
    nv = nv.reshape(B, PW)
    loss = pl.pallas_call(
        _tc_body,
        grid=(B // RB,),
        in_specs=[
            pl.BlockSpec((RB, PW), lambda i: (i, 0)),
            pl.BlockSpec((RB, PW), lambda i: (i, 0)),
        ],
        out_specs=pl.BlockSpec((1, 1), lambda i: (0, 0)),
        out_shape=jax.ShapeDtypeStruct((1, 1), jnp.float32),
    )(dot, nv)
    return loss[0, 0]
